# baseline (speedup 1.0000x reference)
import functools

import jax
import jax.numpy as jnp
from jax import lax
from jax.experimental import pallas as pl
from jax.experimental.pallas import tpu as pltpu
from jax.experimental.pallas import tpu_sc as plsc

VOCAB_ = 100000
HIDDEN_ = 128
CPAD = 4
SEQ_ = 200
BATCH_ = 4096

NC, NS, LANES = 2, 16, 16
UNROLL = 8


def _tc_matmul_body(e_ref, wt_ref, out_ref):
    out_ref[...] = lax.dot_general(
        wt_ref[...], e_ref[...],
        dimension_numbers=(((1,), (1,)), ((), ())),
        preferred_element_type=jnp.float32,
    )


def _project_table(embedding, wst):
    bv = 12544
    grid = pl.cdiv(VOCAB_, bv)
    return pl.pallas_call(
        _tc_matmul_body,
        grid=(grid,),
        in_specs=[
            pl.BlockSpec((bv, HIDDEN_), lambda i: (i, 0)),
            pl.BlockSpec((CPAD, HIDDEN_), lambda i: (0, 0)),
        ],
        out_specs=pl.BlockSpec((CPAD, bv), lambda i: (0, i)),
        out_shape=jax.ShapeDtypeStruct((CPAD, VOCAB_), jnp.float32),
    )(embedding, wst)


def _make_sc_body(n_workers, nslice, batch):
    b_per_w = batch // nslice
    groups = b_per_w // LANES
    chunk = LANES * SEQ_

    def body(ewt_hbm, ids_hbm, out_hbm, tab_v, ids_v0, ids_v1, res_v,
             sem0, sem1):
        wid = lax.axis_index("s") * NC + lax.axis_index("c")
        if n_workers == NS:
            wid = lax.axis_index("s")
        cls = wid // nslice
        sl = wid % nslice
        gbase = sl * groups

        pltpu.sync_copy(ewt_hbm.at[cls], tab_v)

        def fetch(gidx, buf, sem):
            pltpu.make_async_copy(ids_hbm.at[gidx], buf, sem).start()

        def drain(gidx, buf, sem):
            pltpu.make_async_copy(ids_hbm.at[gidx], buf, sem).wait()

        def accumulate(ids_v):
            def seq_body(t, acc):
                for j in range(UNROLL):
                    iv = ids_v[pl.ds((t * UNROLL + j) * LANES, LANES)]
                    acc = acc + plsc.load_gather(tab_v, [iv])
                return acc
            return lax.fori_loop(0, SEQ_ // UNROLL, seq_body,
                                 jnp.zeros((LANES,), jnp.float32))

        fetch(gbase, ids_v0, sem0)

        def group_pair(g2, _):
            g = 2 * g2
            drain(gbase + g, ids_v0, sem0)
            fetch(gbase + lax.rem(g + 1, groups), ids_v1, sem1)
            res_v[pl.ds(g * LANES, LANES)] = accumulate(ids_v0)
            drain(gbase, ids_v1, sem1)
            fetch(gbase + lax.rem(g + 2, groups), ids_v0, sem0)
            res_v[pl.ds((g + 1) * LANES, LANES)] = accumulate(ids_v1)
            return 0

        lax.fori_loop(0, groups // 2, group_pair, 0)
        drain(gbase, ids_v0, sem0)

        pltpu.sync_copy(res_v, out_hbm.at[cls, pl.ds(sl * b_per_w, b_per_w)])

    mesh = plsc.VectorSubcoreMesh(
        core_axis_name="c", subcore_axis_name="s",
        num_cores=n_workers // NS, num_subcores=NS)
    return functools.partial(
        pl.kernel,
        out_type=jax.ShapeDtypeStruct((CPAD, batch), jnp.float32),
        mesh=mesh,
        compiler_params=pltpu.CompilerParams(needs_layout_passes=False),
        scratch_types=[
            pltpu.VMEM((VOCAB_,), jnp.float32),
            pltpu.VMEM((chunk,), jnp.int32),
            pltpu.VMEM((chunk,), jnp.int32),
            pltpu.VMEM((b_per_w,), jnp.float32),
            pltpu.SemaphoreType.DMA,
            pltpu.SemaphoreType.DMA,
        ],
    )(body)


_sc_gather_sum_half = _make_sc_body(n_workers=NS, nslice=NS // CPAD,
                                    batch=BATCH_ // 2)


def kernel(input_ids, embedding, W, b):
    ids3 = (input_ids.astype(jnp.int32)
            .reshape(BATCH_ // LANES, LANES, SEQ_)
            .transpose(0, 2, 1)
            .reshape(BATCH_ // LANES, LANES * SEQ_))
    wst = jnp.pad((W / SEQ_).astype(jnp.float32).T,
                  ((0, CPAD - W.shape[1]), (0, 0)))
    ewt = _project_table(embedding, wst)
    nh = BATCH_ // (2 * LANES)
    sums_a = _sc_gather_sum_half(ewt, ids3[:nh])
    sums_b = _sc_gather_sum_half(ewt, ids3[nh:])
    sums = jnp.concatenate([sums_a, sums_b], axis=1)
    return sums[: W.shape[1]].T + b


# speedup vs baseline: 1.3600x; 1.3600x over previous
import functools

import jax
import jax.numpy as jnp
from jax import lax
from jax.experimental import pallas as pl
from jax.experimental.pallas import tpu as pltpu
from jax.experimental.pallas import tpu_sc as plsc

VOCAB_ = 100000
HIDDEN_ = 128
CPAD = 4
SEQ_ = 200
BATCH_ = 4096

NC, NS, LANES = 2, 16, 16
UNROLL = 8


def _tc_matmul_body(e_ref, wt_ref, out_ref):
    out_ref[...] = lax.dot_general(
        wt_ref[...], e_ref[...],
        dimension_numbers=(((1,), (1,)), ((), ())),
        preferred_element_type=jnp.float32,
    )


def _project_table(embedding, wst):
    bv = 12544
    grid = pl.cdiv(VOCAB_, bv)
    return pl.pallas_call(
        _tc_matmul_body,
        grid=(grid,),
        in_specs=[
            pl.BlockSpec((bv, HIDDEN_), lambda i: (i, 0)),
            pl.BlockSpec((CPAD, HIDDEN_), lambda i: (0, 0)),
        ],
        out_specs=pl.BlockSpec((CPAD, bv), lambda i: (0, i)),
        out_shape=jax.ShapeDtypeStruct((CPAD, VOCAB_), jnp.float32),
    )(embedding, wst)


def _make_sc_body(n_workers, nslice, batch):
    b_per_w = batch // nslice
    groups = b_per_w // LANES
    chunk = LANES * SEQ_

    def body(ewt_hbm, ids_hbm, out_hbm, tab_v, ids_v0, ids_v1, res_v,
             sem0, sem1):
        wid = lax.axis_index("s") * NC + lax.axis_index("c")
        if n_workers == NS:
            wid = lax.axis_index("s")
        cls = wid // nslice
        sl = wid % nslice
        gbase = sl * groups

        pltpu.sync_copy(ewt_hbm.at[cls], tab_v)

        def fetch(gidx, buf, sem):
            pltpu.make_async_copy(ids_hbm.at[gidx], buf, sem).start()

        def drain(gidx, buf, sem):
            pltpu.make_async_copy(ids_hbm.at[gidx], buf, sem).wait()

        def accumulate(ids_v):
            def seq_body(t, acc):
                for j in range(UNROLL):
                    iv = ids_v[pl.ds((t * UNROLL + j) * LANES, LANES)]
                    acc = acc + plsc.load_gather(tab_v, [iv])
                return acc
            return lax.fori_loop(0, SEQ_ // UNROLL, seq_body,
                                 jnp.zeros((LANES,), jnp.float32))

        fetch(gbase, ids_v0, sem0)

        def group_pair(g2, _):
            g = 2 * g2
            drain(gbase + g, ids_v0, sem0)
            fetch(gbase + lax.rem(g + 1, groups), ids_v1, sem1)
            res_v[pl.ds(g * LANES, LANES)] = accumulate(ids_v0)
            drain(gbase, ids_v1, sem1)
            fetch(gbase + lax.rem(g + 2, groups), ids_v0, sem0)
            res_v[pl.ds((g + 1) * LANES, LANES)] = accumulate(ids_v1)
            return 0

        lax.fori_loop(0, groups // 2, group_pair, 0)
        drain(gbase, ids_v0, sem0)

        pltpu.sync_copy(res_v, out_hbm.at[cls, pl.ds(sl * b_per_w, b_per_w)])

    mesh = plsc.VectorSubcoreMesh(
        core_axis_name="c", subcore_axis_name="s",
        num_cores=n_workers // NS, num_subcores=NS)
    return functools.partial(
        pl.kernel,
        out_type=jax.ShapeDtypeStruct((CPAD, batch), jnp.float32),
        mesh=mesh,
        compiler_params=pltpu.CompilerParams(needs_layout_passes=False),
        scratch_types=[
            pltpu.VMEM((VOCAB_,), jnp.float32),
            pltpu.VMEM((chunk,), jnp.int32),
            pltpu.VMEM((chunk,), jnp.int32),
            pltpu.VMEM((b_per_w,), jnp.float32),
            pltpu.SemaphoreType.DMA,
            pltpu.SemaphoreType.DMA,
        ],
    )(body)


_sc_gather_sum = _make_sc_body(n_workers=NC * NS, nslice=NC * NS // CPAD,
                               batch=BATCH_)


def kernel(input_ids, embedding, W, b):
    ids3 = (input_ids.astype(jnp.int32)
            .reshape(BATCH_ // LANES, LANES, SEQ_)
            .transpose(0, 2, 1)
            .reshape(BATCH_ // LANES, LANES * SEQ_))
    wst = jnp.pad((W / SEQ_).astype(jnp.float32).T,
                  ((0, CPAD - W.shape[1]), (0, 0)))
    ewt = _project_table(embedding, wst)
    sums = _sc_gather_sum(ewt, ids3)
    return sums[: W.shape[1]].T + b


# speedup vs baseline: 1.3792x; 1.0141x over previous
import functools

import jax
import jax.numpy as jnp
from jax import lax
from jax.experimental import pallas as pl
from jax.experimental.pallas import tpu as pltpu
from jax.experimental.pallas import tpu_sc as plsc

VOCAB_ = 100000
HIDDEN_ = 128
CPAD = 4
SEQ_ = 200
BATCH_ = 4096

NC, NS, LANES = 2, 16, 16
UNROLL = 8
NACC = 4


def _tc_body(e_ref, wt_ref, ids_ref, out_ref, idst_ref):
    out_ref[...] = lax.dot_general(
        wt_ref[...], e_ref[...],
        dimension_numbers=(((1,), (1,)), ((), ())),
        preferred_element_type=jnp.float32,
    )
    blk = ids_ref[...]
    g = blk.shape[0] // LANES
    idst_ref[...] = (blk.reshape(g, LANES, SEQ_)
                     .transpose(0, 2, 1)
                     .reshape(g, SEQ_ * LANES))


def _project_and_transpose(embedding, wst, ids):
    bv = 12544
    grid = pl.cdiv(VOCAB_, bv)
    bb = BATCH_ // grid
    return pl.pallas_call(
        _tc_body,
        grid=(grid,),
        in_specs=[
            pl.BlockSpec((bv, HIDDEN_), lambda i: (i, 0)),
            pl.BlockSpec((CPAD, HIDDEN_), lambda i: (0, 0)),
            pl.BlockSpec((bb, SEQ_), lambda i: (i, 0)),
        ],
        out_specs=[
            pl.BlockSpec((CPAD, bv), lambda i: (0, i)),
            pl.BlockSpec((bb // LANES, SEQ_ * LANES), lambda i: (i, 0)),
        ],
        out_shape=[
            jax.ShapeDtypeStruct((CPAD, VOCAB_), jnp.float32),
            jax.ShapeDtypeStruct((BATCH_ // LANES, SEQ_ * LANES), jnp.int32),
        ],
    )(embedding, wst, ids)


def _make_sc_body(n_workers, nslice, batch):
    b_per_w = batch // nslice
    groups = b_per_w // LANES

    def body(ewt_hbm, idst_hbm, out_hbm, tab_v, ids_v0, ids_v1, res_v,
             sem0, sem1):
        wid = lax.axis_index("s") * NC + lax.axis_index("c")
        if n_workers == NS:
            wid = lax.axis_index("s")
        cls = wid // nslice
        sl = wid % nslice
        gbase = sl * groups

        pltpu.sync_copy(ewt_hbm.at[cls], tab_v)

        def fetch(gidx, buf, sem):
            pltpu.make_async_copy(idst_hbm.at[gidx], buf, sem).start()

        def drain(buf, sem):
            pltpu.make_async_copy(idst_hbm.at[0], buf, sem).wait()

        def accumulate(ids_v):
            zero = jnp.zeros((LANES,), jnp.float32)

            def seq_body(t, accs):
                new = list(accs)
                for j in range(UNROLL):
                    iv = ids_v[pl.ds((t * UNROLL + j) * LANES, LANES)]
                    new[j % NACC] = new[j % NACC] + plsc.load_gather(
                        tab_v, [iv])
                return tuple(new)

            accs = lax.fori_loop(0, SEQ_ // UNROLL, seq_body, (zero,) * NACC)
            return (accs[0] + accs[1]) + (accs[2] + accs[3])

        fetch(gbase, ids_v0, sem0)

        def group_pair(g2, _):
            g = 2 * g2
            drain(ids_v0, sem0)
            fetch(gbase + lax.rem(g + 1, groups), ids_v1, sem1)
            res_v[pl.ds(g * LANES, LANES)] = accumulate(ids_v0)
            drain(ids_v1, sem1)
            fetch(gbase + lax.rem(g + 2, groups), ids_v0, sem0)
            res_v[pl.ds((g + 1) * LANES, LANES)] = accumulate(ids_v1)
            return 0

        lax.fori_loop(0, groups // 2, group_pair, 0)
        drain(ids_v0, sem0)

        pltpu.sync_copy(res_v, out_hbm.at[cls, pl.ds(sl * b_per_w, b_per_w)])

    mesh = plsc.VectorSubcoreMesh(
        core_axis_name="c", subcore_axis_name="s",
        num_cores=n_workers // NS, num_subcores=NS)
    return functools.partial(
        pl.kernel,
        out_type=jax.ShapeDtypeStruct((CPAD, batch), jnp.float32),
        mesh=mesh,
        compiler_params=pltpu.CompilerParams(needs_layout_passes=False),
        scratch_types=[
            pltpu.VMEM((VOCAB_,), jnp.float32),
            pltpu.VMEM((SEQ_ * LANES,), jnp.int32),
            pltpu.VMEM((SEQ_ * LANES,), jnp.int32),
            pltpu.VMEM((b_per_w,), jnp.float32),
            pltpu.SemaphoreType.DMA,
            pltpu.SemaphoreType.DMA,
        ],
    )(body)


_sc_gather_sum = _make_sc_body(n_workers=NC * NS, nslice=NC * NS // CPAD,
                               batch=BATCH_)


def kernel(input_ids, embedding, W, b):
    ids = input_ids.astype(jnp.int32)
    wst = jnp.pad((W / SEQ_).astype(jnp.float32).T,
                  ((0, CPAD - W.shape[1]), (0, 0)))
    ewt, idst = _project_and_transpose(embedding, wst, ids)
    sums = _sc_gather_sum(ewt, idst)
    return sums[: W.shape[1]].T + b
